# Initial kernel scaffold; baseline (speedup 1.0000x reference)
import functools

import jax
import jax.numpy as jnp
from jax import lax
from jax.experimental import pallas as pl
from jax.experimental.pallas import tpu as pltpu
from jax.experimental.pallas import tpu_sc as plsc

N = 10000
E = 320000
NC, NS = 2, 16
NT = NC * NS
B = 128
EPT = 10240
CH = EPT // B
NP = 10240
ZR = 64
RPT = NP // NS
ORT = N // NS

_MESH = plsc.VectorSubcoreMesh(
    core_axis_name="c", subcore_axis_name="s", num_cores=NC, num_subcores=NS
)


@functools.partial(
    pl.kernel,
    out_type=jax.ShapeDtypeStruct((NC, N, 16), jnp.float32),
    mesh=_MESH,
    scratch_types=[
        pltpu.VMEM((CH, B), jnp.int32),
        pltpu.VMEM((B, 16), jnp.float32),
        pltpu.VMEM((ZR, 16), jnp.float32),
        pltpu.VMEM_SHARED((NP, 16), jnp.float32),
    ],
)
def _sc_degree(dst_hbm, out_hbm, dst_v, ones_v, zbuf, acc):
    c = lax.axis_index("c")
    s = lax.axis_index("s")
    t = c * NS + s

    pltpu.sync_copy(dst_hbm.at[t], dst_v)

    @pl.loop(0, B)
    def _(i):
        ones_v[i, :] = jnp.ones((16,), jnp.float32)

    @pl.loop(0, ZR)
    def _(i):
        zbuf[i, :] = jnp.zeros((16,), jnp.float32)

    @pl.loop(0, RPT, step=ZR)
    def _(r):
        pltpu.sync_copy(zbuf, acc.at[pl.ds(s * RPT + r, ZR)])

    plsc.subcore_barrier()

    @pl.loop(0, CH)
    def _(ch):
        pltpu.sync_copy(ones_v, acc.at[dst_v.at[ch]], add=True)

    plsc.subcore_barrier()
    pltpu.sync_copy(acc.at[pl.ds(s * ORT, ORT)], out_hbm.at[c, pl.ds(s * ORT, ORT)])


def _make_sc_gather_scatter(d):
    @functools.partial(
        pl.kernel,
        out_type=jax.ShapeDtypeStruct((NC, N, d), jnp.float32),
        mesh=_MESH,
        scratch_types=[
            pltpu.VMEM((CH, B), jnp.int32),
            pltpu.VMEM((CH, B), jnp.int32),
            pltpu.VMEM((B, d), jnp.float32),
            pltpu.VMEM((B, d), jnp.float32),
            pltpu.VMEM((ZR, d), jnp.float32),
            pltpu.VMEM_SHARED((NP, d), jnp.float32),
            pltpu.SemaphoreType.DMA,
            pltpu.SemaphoreType.DMA,
            pltpu.SemaphoreType.DMA,
            pltpu.SemaphoreType.DMA,
        ],
    )
    def sc_kernel(src_hbm, dst_hbm, g_hbm, out_hbm,
                  src_v, dst_v, buf0, buf1, zbuf, acc, gs0, gs1, ss0, ss1):
        c = lax.axis_index("c")
        s = lax.axis_index("s")
        t = c * NS + s

        pltpu.sync_copy(src_hbm.at[t], src_v)
        pltpu.sync_copy(dst_hbm.at[t], dst_v)

        @pl.loop(0, ZR)
        def _(i):
            @pl.loop(0, d, step=16)
            def _(j):
                zbuf[i, pl.ds(j, 16)] = jnp.zeros((16,), jnp.float32)

        @pl.loop(0, RPT, step=ZR)
        def _(r):
            pltpu.sync_copy(zbuf, acc.at[pl.ds(s * RPT + r, ZR)])

        plsc.subcore_barrier()

        pltpu.async_copy(g_hbm.at[src_v.at[0]], buf0, gs0)
        pltpu.async_copy(g_hbm.at[src_v.at[1]], buf1, gs1)

        @pl.loop(0, CH, step=2)
        def _(ch):
            pltpu.make_async_copy(g_hbm.at[src_v.at[ch]], buf0, gs0).wait()
            pltpu.async_copy(buf0, acc.at[dst_v.at[ch]], ss0, add=True)
            pltpu.make_async_copy(g_hbm.at[src_v.at[ch]], buf1, gs1).wait()
            pltpu.async_copy(buf1, acc.at[dst_v.at[ch + 1]], ss1, add=True)

            pltpu.make_async_copy(buf0, acc.at[dst_v.at[ch]], ss0).wait()

            @pl.when(ch + 2 < CH)
            def _():
                pltpu.async_copy(g_hbm.at[src_v.at[ch + 2]], buf0, gs0)

            pltpu.make_async_copy(buf1, acc.at[dst_v.at[ch]], ss1).wait()

            @pl.when(ch + 3 < CH)
            def _():
                pltpu.async_copy(g_hbm.at[src_v.at[ch + 3]], buf1, gs1)

        plsc.subcore_barrier()
        pltpu.sync_copy(acc.at[pl.ds(s * ORT, ORT)],
                        out_hbm.at[c, pl.ds(s * ORT, ORT)])

    return sc_kernel


_sc_gs_128 = _make_sc_gather_scatter(128)
_sc_gs_64 = _make_sc_gather_scatter(64)


_R = 2000


def _tc_matmul(x, w):
    def body(x_ref, w_ref, o_ref):
        o_ref[...] = jnp.dot(x_ref[...], w_ref[...],
                             preferred_element_type=jnp.float32)

    return pl.pallas_call(
        body,
        grid=(N // _R,),
        in_specs=[
            pl.BlockSpec((_R, x.shape[1]), lambda i: (i, 0)),
            pl.BlockSpec(w.shape, lambda i: (0, 0)),
        ],
        out_specs=pl.BlockSpec((_R, w.shape[1]), lambda i: (i, 0)),
        out_shape=jax.ShapeDtypeStruct((N, w.shape[1]), jnp.float32),
    )(x, w)


def _tc_scale(cnt, h):
    d = h.shape[1]

    def body(cnt_ref, h_ref, o_ref):
        dinv = lax.rsqrt(1.0 + cnt_ref[0, :, 0:1] + cnt_ref[1, :, 0:1])
        o_ref[...] = h_ref[...] * dinv

    return pl.pallas_call(
        body,
        grid=(N // _R,),
        in_specs=[
            pl.BlockSpec((NC, _R, 16), lambda i: (0, i, 0)),
            pl.BlockSpec((_R, d), lambda i: (i, 0)),
        ],
        out_specs=pl.BlockSpec((_R, d), lambda i: (i, 0)),
        out_shape=jax.ShapeDtypeStruct((N, d), jnp.float32),
    )(cnt, h)


def _tc_combine_next(cnt, acc, g, b, w):
    d = g.shape[1]
    dn = w.shape[1]

    def body(cnt_ref, acc_ref, g_ref, b_ref, w_ref, o_ref):
        dinv = lax.rsqrt(1.0 + cnt_ref[0, :, 0:1] + cnt_ref[1, :, 0:1])
        t = (acc_ref[0] + acc_ref[1] + g_ref[...]) * dinv
        t = t + jnp.reshape(b_ref[...], (1, d))
        t = jnp.maximum(t, 0.0)
        o_ref[...] = jnp.dot(t, w_ref[...],
                             preferred_element_type=jnp.float32) * dinv

    return pl.pallas_call(
        body,
        grid=(N // _R,),
        in_specs=[
            pl.BlockSpec((NC, _R, 16), lambda i: (0, i, 0)),
            pl.BlockSpec((NC, _R, d), lambda i: (0, i, 0)),
            pl.BlockSpec((_R, d), lambda i: (i, 0)),
            pl.BlockSpec((d,), lambda i: (0,)),
            pl.BlockSpec((d, dn), lambda i: (0, 0)),
        ],
        out_specs=pl.BlockSpec((_R, dn), lambda i: (i, 0)),
        out_shape=jax.ShapeDtypeStruct((N, dn), jnp.float32),
    )(cnt, acc, g, b, w)


def _tc_final(cnt, acc, g, b):
    d = g.shape[1]

    def body(cnt_ref, acc_ref, g_ref, b_ref, o_ref):
        dinv = lax.rsqrt(1.0 + cnt_ref[0, :, 0:1] + cnt_ref[1, :, 0:1])
        t = (acc_ref[0] + acc_ref[1] + g_ref[...]) * dinv
        o_ref[...] = t + jnp.reshape(b_ref[...], (1, d))

    return pl.pallas_call(
        body,
        grid=(N // _R,),
        in_specs=[
            pl.BlockSpec((NC, _R, 16), lambda i: (0, i, 0)),
            pl.BlockSpec((NC, _R, d), lambda i: (0, i, 0)),
            pl.BlockSpec((_R, d), lambda i: (i, 0)),
            pl.BlockSpec((d,), lambda i: (0,)),
        ],
        out_specs=pl.BlockSpec((_R, d), lambda i: (i, 0)),
        out_shape=jax.ShapeDtypeStruct((N, d), jnp.float32),
    )(cnt, acc, g, b)


@jax.jit
def kernel(x, edge_index, W1, b1, W2, b2, W3, b3):
    pad = NT * EPT - E
    src = jnp.concatenate(
        [edge_index[0], jnp.zeros((pad,), jnp.int32)]).reshape(NT, CH, B)
    dst = jnp.concatenate(
        [edge_index[1], jnp.full((pad,), N, jnp.int32)]).reshape(NT, CH, B)

    cnt = _sc_degree(dst)

    h1 = _tc_matmul(x, W1)
    g1 = _tc_scale(cnt, h1)

    acc1 = _sc_gs_128(src, dst, g1)
    g2 = _tc_combine_next(cnt, acc1, g1, b1, W2)

    acc2 = _sc_gs_128(src, dst, g2)
    g3 = _tc_combine_next(cnt, acc2, g2, b2, W3)

    acc3 = _sc_gs_64(src, dst, g3)
    return _tc_final(cnt, acc3, g3, b3)


# speedup vs baseline: 12.1345x; 12.1345x over previous
import functools

import jax
import jax.numpy as jnp
from jax import lax
from jax.experimental import pallas as pl
from jax.experimental.pallas import tpu as pltpu
from jax.experimental.pallas import tpu_sc as plsc

N = 10000
E = 320000
NC, NS = 2, 16
B = 128
EPS = 20480
CHT = EPS // B
CHH = CHT // 2
NP = 10240
ZR = 64
RPT = NP // NS
D2 = 64

_MESH = plsc.VectorSubcoreMesh(
    core_axis_name="c", subcore_axis_name="s", num_cores=NC, num_subcores=NS
)


@functools.partial(
    pl.kernel,
    out_type=jax.ShapeDtypeStruct((NC, NP, 16), jnp.float32),
    mesh=_MESH,
    compiler_params=pltpu.CompilerParams(use_tc_tiling_on_sc=False),
    scratch_types=[
        pltpu.VMEM((CHH, B), jnp.int32),
        pltpu.VMEM((B, 16), jnp.float32),
        pltpu.VMEM((ZR, 16), jnp.float32),
        pltpu.VMEM_SHARED((NP, 16), jnp.float32),
    ],
)
def _sc_degree(dst_hbm, out_hbm, dst_v, ones_v, zbuf, acc):
    c = lax.axis_index("c")
    s = lax.axis_index("s")

    pltpu.sync_copy(dst_hbm.at[s, pl.ds(c * CHH, CHH)], dst_v)

    @pl.loop(0, B)
    def _(i):
        ones_v[i, :] = jnp.ones((16,), jnp.float32)

    @pl.loop(0, ZR)
    def _(i):
        zbuf[i, :] = jnp.zeros((16,), jnp.float32)

    @pl.loop(0, RPT, step=ZR)
    def _(r):
        pltpu.sync_copy(zbuf, acc.at[pl.ds(s * RPT + r, ZR)])

    plsc.subcore_barrier()

    @pl.loop(0, CHH)
    def _(ch):
        pltpu.sync_copy(ones_v, acc.at[dst_v.at[ch]], add=True)

    plsc.subcore_barrier()
    pltpu.sync_copy(acc.at[pl.ds(s * RPT, RPT)],
                    out_hbm.at[c, pl.ds(s * RPT, RPT)])


def _make_sc_gather_scatter(split_cols):
    nch = CHT if split_cols else CHH
    g_row_len = B

    @functools.partial(
        pl.kernel,
        out_type=jax.ShapeDtypeStruct((NC, NP, D2), jnp.float32),
        mesh=_MESH,
        compiler_params=pltpu.CompilerParams(use_tc_tiling_on_sc=False),
        scratch_types=[
            pltpu.VMEM((nch, B), jnp.int32),
            pltpu.VMEM((nch, B), jnp.int32),
            pltpu.VMEM((B, D2), jnp.float32),
            pltpu.VMEM((B, D2), jnp.float32),
            pltpu.VMEM((ZR, D2), jnp.float32),
            pltpu.VMEM_SHARED((NP, D2), jnp.float32),
            pltpu.SemaphoreType.DMA,
            pltpu.SemaphoreType.DMA,
            pltpu.SemaphoreType.DMA,
            pltpu.SemaphoreType.DMA,
        ],
    )
    def sc_kernel(src_hbm, dst_hbm, g_hbm, out_hbm,
                  src_v, dst_v, buf0, buf1, zbuf, acc, gs0, gs1, ss0, ss1):
        c = lax.axis_index("c")
        s = lax.axis_index("s")

        if split_cols:
            pltpu.sync_copy(src_hbm.at[s], src_v)
            pltpu.sync_copy(dst_hbm.at[s], dst_v)
            table = g_hbm.at[c]
        else:
            pltpu.sync_copy(src_hbm.at[s, pl.ds(c * CHH, CHH)], src_v)
            pltpu.sync_copy(dst_hbm.at[s, pl.ds(c * CHH, CHH)], dst_v)
            table = g_hbm

        @pl.loop(0, ZR)
        def _(i):
            @pl.loop(0, D2, step=16)
            def _(j):
                zbuf[i, pl.ds(j, 16)] = jnp.zeros((16,), jnp.float32)

        @pl.loop(0, RPT, step=ZR)
        def _(r):
            pltpu.sync_copy(zbuf, acc.at[pl.ds(s * RPT + r, ZR)])

        plsc.subcore_barrier()

        pltpu.async_copy(table.at[src_v.at[0]], buf0, gs0)
        pltpu.async_copy(table.at[src_v.at[1]], buf1, gs1)

        @pl.loop(0, nch, step=2)
        def _(ch):
            pltpu.make_async_copy(table.at[src_v.at[ch]], buf0, gs0).wait()
            pltpu.async_copy(buf0, acc.at[dst_v.at[ch]], ss0, add=True)
            pltpu.make_async_copy(table.at[src_v.at[ch]], buf1, gs1).wait()
            pltpu.async_copy(buf1, acc.at[dst_v.at[ch + 1]], ss1, add=True)

            pltpu.make_async_copy(buf0, acc.at[dst_v.at[ch]], ss0).wait()

            @pl.when(ch + 2 < nch)
            def _():
                pltpu.async_copy(table.at[src_v.at[ch + 2]], buf0, gs0)

            pltpu.make_async_copy(buf1, acc.at[dst_v.at[ch]], ss1).wait()

            @pl.when(ch + 3 < nch)
            def _():
                pltpu.async_copy(table.at[src_v.at[ch + 3]], buf1, gs1)

        plsc.subcore_barrier()
        pltpu.sync_copy(acc.at[pl.ds(s * RPT, RPT)],
                        out_hbm.at[c, pl.ds(s * RPT, RPT)])

    return sc_kernel


_sc_gs_cols = _make_sc_gather_scatter(True)
_sc_gs_edges = _make_sc_gather_scatter(False)


_R = 2000

_CNT_SPEC = pl.BlockSpec((NC, _R, 16), lambda i: (0, i, 0))


def _tc_matmul(x, w):
    def body(x_ref, w_ref, o_ref):
        o_ref[...] = jnp.dot(x_ref[...], w_ref[...],
                             preferred_element_type=jnp.float32)

    return pl.pallas_call(
        body,
        grid=(N // _R,),
        in_specs=[
            pl.BlockSpec((_R, x.shape[1]), lambda i: (i, 0)),
            pl.BlockSpec(w.shape, lambda i: (0, 0)),
        ],
        out_specs=pl.BlockSpec((_R, w.shape[1]), lambda i: (i, 0)),
        out_shape=jax.ShapeDtypeStruct((N, w.shape[1]), jnp.float32),
    )(x, w)


def _tc_scale(cnt, h):
    def body(cnt_ref, h_ref, o_ref):
        dinv = lax.rsqrt(1.0 + cnt_ref[0, :, 0:1] + cnt_ref[1, :, 0:1])
        g = h_ref[...] * dinv
        o_ref[0] = g[:, :D2]
        o_ref[1] = g[:, D2:]

    return pl.pallas_call(
        body,
        grid=(N // _R,),
        in_specs=[
            _CNT_SPEC,
            pl.BlockSpec((_R, 2 * D2), lambda i: (i, 0)),
        ],
        out_specs=pl.BlockSpec((NC, _R, D2), lambda i: (0, i, 0)),
        out_shape=jax.ShapeDtypeStruct((NC, N, D2), jnp.float32),
    )(cnt, h)


def _tc_combine12(cnt, acc, g, b, w):
    def body(cnt_ref, acc_ref, g_ref, b_ref, w_ref, o_ref):
        dinv = lax.rsqrt(1.0 + cnt_ref[0, :, 0:1] + cnt_ref[1, :, 0:1])
        t = jnp.concatenate(
            [acc_ref[0] + g_ref[0], acc_ref[1] + g_ref[1]], axis=1)
        t = t * dinv + jnp.reshape(b_ref[...], (1, 2 * D2))
        t = jnp.maximum(t, 0.0)
        r = jnp.dot(t, w_ref[...], preferred_element_type=jnp.float32) * dinv
        o_ref[0] = r[:, :D2]
        o_ref[1] = r[:, D2:]

    return pl.pallas_call(
        body,
        grid=(N // _R,),
        in_specs=[
            _CNT_SPEC,
            pl.BlockSpec((NC, _R, D2), lambda i: (0, i, 0)),
            pl.BlockSpec((NC, _R, D2), lambda i: (0, i, 0)),
            pl.BlockSpec((2 * D2,), lambda i: (0,)),
            pl.BlockSpec((2 * D2, 2 * D2), lambda i: (0, 0)),
        ],
        out_specs=pl.BlockSpec((NC, _R, D2), lambda i: (0, i, 0)),
        out_shape=jax.ShapeDtypeStruct((NC, N, D2), jnp.float32),
    )(cnt, acc, g, b, w)


def _tc_combine23(cnt, acc, g, b, w):
    def body(cnt_ref, acc_ref, g_ref, b_ref, w_ref, o_ref):
        dinv = lax.rsqrt(1.0 + cnt_ref[0, :, 0:1] + cnt_ref[1, :, 0:1])
        t = jnp.concatenate(
            [acc_ref[0] + g_ref[0], acc_ref[1] + g_ref[1]], axis=1)
        t = t * dinv + jnp.reshape(b_ref[...], (1, 2 * D2))
        t = jnp.maximum(t, 0.0)
        o_ref[...] = jnp.dot(t, w_ref[...],
                             preferred_element_type=jnp.float32) * dinv

    return pl.pallas_call(
        body,
        grid=(N // _R,),
        in_specs=[
            _CNT_SPEC,
            pl.BlockSpec((NC, _R, D2), lambda i: (0, i, 0)),
            pl.BlockSpec((NC, _R, D2), lambda i: (0, i, 0)),
            pl.BlockSpec((2 * D2,), lambda i: (0,)),
            pl.BlockSpec((2 * D2, D2), lambda i: (0, 0)),
        ],
        out_specs=pl.BlockSpec((_R, D2), lambda i: (i, 0)),
        out_shape=jax.ShapeDtypeStruct((N, D2), jnp.float32),
    )(cnt, acc, g, b, w)


def _tc_final(cnt, acc, g, b):
    def body(cnt_ref, acc_ref, g_ref, b_ref, o_ref):
        dinv = lax.rsqrt(1.0 + cnt_ref[0, :, 0:1] + cnt_ref[1, :, 0:1])
        t = (acc_ref[0] + acc_ref[1] + g_ref[...]) * dinv
        o_ref[...] = t + jnp.reshape(b_ref[...], (1, D2))

    return pl.pallas_call(
        body,
        grid=(N // _R,),
        in_specs=[
            _CNT_SPEC,
            pl.BlockSpec((NC, _R, D2), lambda i: (0, i, 0)),
            pl.BlockSpec((_R, D2), lambda i: (i, 0)),
            pl.BlockSpec((D2,), lambda i: (0,)),
        ],
        out_specs=pl.BlockSpec((_R, D2), lambda i: (i, 0)),
        out_shape=jax.ShapeDtypeStruct((N, D2), jnp.float32),
    )(cnt, acc, g, b)


@jax.jit
def kernel(x, edge_index, W1, b1, W2, b2, W3, b3):
    pad = NS * EPS - E
    src = jnp.concatenate(
        [edge_index[0], jnp.zeros((pad,), jnp.int32)]).reshape(NS, CHT, B)
    dst = jnp.concatenate(
        [edge_index[1], jnp.full((pad,), N, jnp.int32)]).reshape(NS, CHT, B)

    cnt = _sc_degree(dst)

    h1 = _tc_matmul(x, W1)
    g1 = _tc_scale(cnt, h1)

    acc1 = _sc_gs_cols(src, dst, g1)
    g2 = _tc_combine12(cnt, acc1, g1, b1, W2)

    acc2 = _sc_gs_cols(src, dst, g2)
    g3 = _tc_combine23(cnt, acc2, g2, b2, W3)

    acc3 = _sc_gs_edges(src, dst, g3)
    return _tc_final(cnt, acc3, g3, b3)
